# baseline (speedup 1.0000x reference)
import functools

import jax
import jax.numpy as jnp
from jax import lax
from jax.experimental import pallas as pl
from jax.experimental.pallas import tpu as pltpu
from jax.experimental.pallas import tpu_sc as plsc

N = 10000
E = 160000
D = 256

NC = 2
NS = 16
L = 16
NW = NC * NS

NCHUNK = D // L
NR = 320
NPAD = NW * NR
ACC_R = NR + 1
SCB = 2000
NSB = E // SCB
CAP = SCB + L
GB = 80
PACK = 16384

EB = 80
EPT = E // NS
NBLK = EPT // EB

_MESH = plsc.VectorSubcoreMesh(core_axis_name="c", subcore_axis_name="s")


@functools.partial(
    pl.kernel,
    out_type=[
        jax.ShapeDtypeStruct((NPAD, D), jnp.float32),
        jax.ShapeDtypeStruct((NPAD,), jnp.float32),
    ],
    mesh=_MESH,
    scratch_types=[
        pltpu.VMEM((GB, D // 2), jnp.int32),
        pltpu.VMEM((ACC_R, D), jnp.float32),
        pltpu.VMEM((NR + L,), jnp.float32),
        pltpu.SMEM((ACC_R,), jnp.float32),
        pltpu.VMEM((CAP,), jnp.int32),
        pltpu.VMEM((GB,), jnp.int32),
        pltpu.VMEM((SCB,), jnp.int32),
        pltpu.VMEM((SCB,), jnp.int32),
        pltpu.SemaphoreType.DMA,
    ],
)
def _seg_sum(x_hbm, bei_hbm, agg_hbm, deg_hbm,
             rows_v, acc_v, degv, dega_sm, idxc, sg_v, src_v, dst_v, sem):
    c = lax.axis_index("c")
    s = lax.axis_index("s")
    w = c * NS + s
    lo = w * NR

    zero = jnp.zeros((L,), jnp.float32)
    zero_i = jnp.zeros((L,), jnp.int32)
    one = jnp.ones((L,), jnp.float32)
    ones_i = jnp.ones((L,), jnp.int32)

    def z_acc(i, _):
        for j in range(NCHUNK):
            acc_v[i, pl.ds(j * L, L)] = zero
        dega_sm[i] = 0.0
        return 0

    lax.fori_loop(0, ACC_R, z_acc, 0)

    def z_deg(i, _):
        degv[pl.ds(i * L, L)] = zero
        return 0

    lax.fori_loop(0, (NR + L) // L, z_deg, 0)

    def z_idx(i, _):
        idxc[pl.ds(i * L, L)] = zero_i
        return 0

    lax.fori_loop(0, CAP // L, z_idx, 0)
    for k2 in range(GB // L):
        sg_v[pl.ds(k2 * L, L)] = zero_i

    def sblk(b, _):
        ebase = b * SCB
        pltpu.sync_copy(bei_hbm.at[pl.ds(ebase, SCB)], src_v)
        pltpu.sync_copy(bei_hbm.at[pl.ds(E + ebase, SCB)], dst_v)

        def chunk(t, cnt):
            s16 = src_v[pl.ds(t * L, L)]
            d16 = dst_v[pl.ds(t * L, L)]
            okv = (d16 >= lo) & (d16 < lo + NR)
            packed = (s16 * PACK + d16) * 2 + jnp.where(okv, 1, 0)
            for k in range(L):
                pk = packed[k]
                idxc[pl.ds(cnt, L)] = ones_i * pk
                cnt = cnt + (pk & 1)
            return cnt

        cnt = lax.fori_loop(0, SCB // L, chunk, jnp.int32(0))

        nb = (cnt + (GB - 1)) // GB

        def gblk(bb, _):
            for k2 in range(GB // L):
                cb0 = idxc[pl.ds(bb * GB + k2 * L, L)]
                sg_v[pl.ds(k2 * L, L)] = jnp.right_shift(cb0, 15)
            pltpu.async_copy(x_hbm.at[sg_v], rows_v, sem).wait()

            def grp(q, _):
                gbase = bb * GB + q * L
                cb = idxc[pl.ds(gbase, L)]
                d16 = jnp.bitwise_and(jnp.right_shift(cb, 1), PACK - 1)
                for k in range(L):
                    e = gbase + k
                    row = jnp.where(e < cnt, d16[k] - lo, NR)
                    er = q * L + k
                    for j in range(NCHUNK // 2):
                        sl = pl.ds(j * L, L)
                        w32 = rows_v[er, sl]
                        wlo = jax.lax.bitcast_convert_type(
                            jax.lax.shift_left(w32, 16), jnp.float32)
                        whi = jax.lax.bitcast_convert_type(
                            jnp.bitwise_and(w32, -65536), jnp.float32)
                        slh = pl.ds((NCHUNK // 2 + j) * L, L)
                        acc_v[row, sl] = acc_v[row, sl] + wlo
                        acc_v[row, slh] = acc_v[row, slh] + whi
                    dega_sm[row] = dega_sm[row] + 1.0
                return 0

            lax.fori_loop(0, GB // L, grp, 0)
            return 0

        lax.fori_loop(0, nb, gblk, 0)
        return 0

    lax.fori_loop(0, NSB, sblk, 0)

    def fin(i, _):
        degv[pl.ds(i, L)] = one * dega_sm[i]
        return 0

    lax.fori_loop(0, NR, fin, 0)
    pltpu.sync_copy(acc_v.at[pl.ds(0, NR)], agg_hbm.at[pl.ds(w * NR, NR)])
    pltpu.sync_copy(degv.at[pl.ds(0, NR)], deg_hbm.at[pl.ds(w * NR, NR)])


def _emb_body(x_ref, agg_ref, deg_ref, ws_ref, wn_ref, h_ref):
    deg = deg_ref[...]
    scale = 1.0 / jnp.maximum(deg, 1.0)
    a = agg_ref[...] * scale
    h = jnp.dot(x_ref[...], ws_ref[...], preferred_element_type=jnp.float32)
    h = h + jnp.dot(a, wn_ref[...], preferred_element_type=jnp.float32)
    h_ref[...] = jnp.maximum(h, 0.0).astype(jnp.bfloat16)


_ROWS_BLK = 1000


def _emb(x, agg, degw, W_self, W_neigh):
    return pl.pallas_call(
        _emb_body,
        grid=(N // _ROWS_BLK,),
        in_specs=[
            pl.BlockSpec((_ROWS_BLK, D), lambda i: (i, 0)),
            pl.BlockSpec((_ROWS_BLK, D), lambda i: (i, 0)),
            pl.BlockSpec((_ROWS_BLK, 1), lambda i: (i, 0)),
            pl.BlockSpec((D, D), lambda i: (0, 0)),
            pl.BlockSpec((D, D), lambda i: (0, 0)),
        ],
        out_specs=pl.BlockSpec((_ROWS_BLK, D), lambda i: (i, 0)),
        out_shape=jax.ShapeDtypeStruct((N, D), jnp.bfloat16),
    )(x, agg, degw, W_self, W_neigh)


DH = D // 2


@functools.partial(
    pl.kernel,
    out_type=[
        jax.ShapeDtypeStruct((E,), jnp.float32),
        jax.ShapeDtypeStruct((E,), jnp.float32),
    ],
    mesh=_MESH,
    scratch_types=[
        pltpu.VMEM((EB, DH), jnp.int32),
        pltpu.VMEM((EB, DH), jnp.int32),
        pltpu.VMEM((EB, DH), jnp.int32),
        pltpu.VMEM((EB, DH), jnp.int32),
        pltpu.VMEM((EB,), jnp.int32),
        pltpu.VMEM((EB,), jnp.int32),
        pltpu.VMEM((EB,), jnp.int32),
        pltpu.VMEM((EB,), jnp.int32),
        pltpu.VMEM((DH,), jnp.float32),
        pltpu.VMEM((DH,), jnp.float32),
        pltpu.VMEM((EPT,), jnp.float32),
        pltpu.SemaphoreType.DMA,
        pltpu.SemaphoreType.DMA,
        pltpu.SemaphoreType.DMA,
        pltpu.SemaphoreType.DMA,
    ],
)
def _score(h_hbm, pos_hbm, neg_hbm, re_hbm, ro_hbm, pos_out, neg_out,
           ua, va, ub, vb, uia, via, uib, vib, re_v, ro_v, sbuf,
           sua, sva, sub_, svb):
    c = lax.axis_index("c")
    s = lax.axis_index("s")

    pltpu.sync_copy(re_hbm, re_v)
    pltpu.sync_copy(ro_hbm, ro_v)
    re_regs = [re_v[pl.ds(j * L, L)] for j in range(DH // L)]
    ro_regs = [ro_v[pl.ds(j * L, L)] for j in range(DH // L)]
    lane = lax.iota(jnp.int32, L)
    onehots = [jnp.where(lane == k, 1.0, 0.0) for k in range(L)]

    def do_set(ei_hbm, out_hbm):
        def fire(b, ui, vi, ur, vr, su, sv):
            base = s * EPT + b * EB
            pltpu.sync_copy(ei_hbm.at[pl.ds(base, EB)], ui)
            pltpu.sync_copy(ei_hbm.at[pl.ds(E + base, EB)], vi)
            cu = pltpu.async_copy(h_hbm.at[ui], ur, su)
            cv = pltpu.async_copy(h_hbm.at[vi], vr, sv)
            return cu, cv

        def compute(b, ur, vr):
            def grp(q, _):
                vec = jnp.zeros((L,), jnp.float32)
                for k in range(L):
                    e = q * L + k
                    acc = None
                    for j in range(DH // L):
                        sl = pl.ds(j * L, L)
                        uw = ur[e, sl]
                        vw = vr[e, sl]
                        ulo = jax.lax.bitcast_convert_type(
                            jax.lax.shift_left(uw, 16), jnp.float32)
                        uhi = jax.lax.bitcast_convert_type(uw, jnp.float32)
                        vlo = jax.lax.bitcast_convert_type(
                            jax.lax.shift_left(vw, 16), jnp.float32)
                        vhi = jax.lax.bitcast_convert_type(vw, jnp.float32)
                        t = ulo * vlo * re_regs[j] + uhi * vhi * ro_regs[j]
                        acc = t if acc is None else acc + t
                    lanes = [acc[i] for i in range(L)]
                    while len(lanes) > 1:
                        lanes = [lanes[i] + lanes[i + 1]
                                 for i in range(0, len(lanes), 2)]
                    vec = vec + lanes[0] * onehots[k]
                sbuf[pl.ds(b * EB + q * L, L)] = vec
                return 0

            lax.fori_loop(0, EB // L, grp, 0)

        ca = fire(0, uia, via, ua, va, sua, sva)

        def wait(ur, su):
            pltpu.make_async_copy(h_hbm.at[uia], ur, su).wait()

        def pair(i, _):
            b = 2 * i
            wait(ua, sua)
            wait(va, sva)
            fire(b + 1, uib, vib, ub, vb, sub_, svb)
            compute(b, ua, va)
            wait(ub, sub_)
            wait(vb, svb)
            fire(b + 2, uia, via, ua, va, sua, sva)
            compute(b + 1, ub, vb)
            return 0

        lax.fori_loop(0, (NBLK - 1) // 2, pair, 0)
        wait(ua, sua)
        wait(va, sva)
        compute(NBLK - 1, ua, va)
        pltpu.sync_copy(sbuf, out_hbm.at[pl.ds(s * EPT, EPT)])

    @pl.when(c == 0)
    def _():
        do_set(pos_hbm, pos_out)

    @pl.when(c == 1)
    def _():
        do_set(neg_hbm, neg_out)


def kernel(x, block_edge_index, pos_edge_index, neg_edge_index, W_self, W_neigh, r):
    x32 = jax.lax.bitcast_convert_type(
        x.astype(jnp.bfloat16).reshape(N, D // 2, 2), jnp.int32)
    agg, degw = _seg_sum(x32, block_edge_index.reshape(-1))
    Wn_perm = jnp.concatenate([W_neigh[0::2], W_neigh[1::2]], axis=0)
    h = _emb(x, agg[:N], degw[:N].reshape(N, 1), W_self, Wn_perm)
    h32 = jax.lax.bitcast_convert_type(h.reshape(N, DH, 2), jnp.int32)
    pos_score, neg_score = _score(h32, pos_edge_index.reshape(-1),
                                  neg_edge_index.reshape(-1),
                                  r[0::2], r[1::2])
    return (pos_score, neg_score)


# speedup vs baseline: 1.0674x; 1.0674x over previous
import functools

import jax
import jax.numpy as jnp
from jax import lax
from jax.experimental import pallas as pl
from jax.experimental.pallas import tpu as pltpu
from jax.experimental.pallas import tpu_sc as plsc

N = 10000
E = 160000
D = 256

NC = 2
NS = 16
L = 16
NW = NC * NS

NCHUNK = D // L
NR = 320
NPAD = NW * NR
ACC_R = NR + 1
SCB = 8000
NSB = E // SCB
CAP = SCB + L
GB = 80
PACK = 16384

EB = 80
EPT = E // NS
NBLK = EPT // EB

_MESH = plsc.VectorSubcoreMesh(core_axis_name="c", subcore_axis_name="s")


@functools.partial(
    pl.kernel,
    out_type=[
        jax.ShapeDtypeStruct((NPAD, D), jnp.float32),
        jax.ShapeDtypeStruct((NPAD,), jnp.float32),
    ],
    mesh=_MESH,
    scratch_types=[
        pltpu.VMEM((GB, D // 2), jnp.int32),
        pltpu.VMEM((ACC_R, D), jnp.float32),
        pltpu.VMEM((NR + L,), jnp.float32),
        pltpu.SMEM((ACC_R,), jnp.float32),
        pltpu.VMEM((CAP,), jnp.int32),
        pltpu.VMEM((GB,), jnp.int32),
        pltpu.VMEM((SCB,), jnp.int32),
        pltpu.VMEM((SCB,), jnp.int32),
        pltpu.SemaphoreType.DMA,
    ],
)
def _seg_sum(x_hbm, bei_hbm, agg_hbm, deg_hbm,
             rows_v, acc_v, degv, dega_sm, idxc, sg_v, src_v, dst_v, sem):
    c = lax.axis_index("c")
    s = lax.axis_index("s")
    w = c * NS + s
    lo = w * NR

    zero = jnp.zeros((L,), jnp.float32)
    zero_i = jnp.zeros((L,), jnp.int32)
    one = jnp.ones((L,), jnp.float32)
    ones_i = jnp.ones((L,), jnp.int32)

    def z_acc(i, _):
        for j in range(NCHUNK):
            acc_v[i, pl.ds(j * L, L)] = zero
        dega_sm[i] = 0.0
        return 0

    lax.fori_loop(0, ACC_R, z_acc, 0)

    def z_deg(i, _):
        degv[pl.ds(i * L, L)] = zero
        return 0

    lax.fori_loop(0, (NR + L) // L, z_deg, 0)

    def z_idx(i, _):
        idxc[pl.ds(i * L, L)] = zero_i
        return 0

    lax.fori_loop(0, CAP // L, z_idx, 0)
    for k2 in range(GB // L):
        sg_v[pl.ds(k2 * L, L)] = zero_i

    def sblk(b, _):
        ebase = b * SCB
        pltpu.sync_copy(bei_hbm.at[pl.ds(ebase, SCB)], src_v)
        pltpu.sync_copy(bei_hbm.at[pl.ds(E + ebase, SCB)], dst_v)

        def chunk(t, cnt):
            s16 = src_v[pl.ds(t * L, L)]
            d16 = dst_v[pl.ds(t * L, L)]
            comb = s16 * PACK + d16
            okv = (d16 >= lo) & (d16 < lo + NR)
            oki = jnp.where(okv, 1, 0)
            for k in range(L):
                idxc[pl.ds(cnt, L)] = ones_i * comb[k]
                cnt = cnt + oki[k]
            return cnt

        cnt = lax.fori_loop(0, SCB // L, chunk, jnp.int32(0))

        nb = (cnt + (GB - 1)) // GB

        def gblk(bb, _):
            for k2 in range(GB // L):
                cb0 = idxc[pl.ds(bb * GB + k2 * L, L)]
                sg_v[pl.ds(k2 * L, L)] = jnp.right_shift(cb0, 14)
            pltpu.async_copy(x_hbm.at[sg_v], rows_v, sem).wait()

            def grp(q, _):
                gbase = bb * GB + q * L
                cb = idxc[pl.ds(gbase, L)]
                d16 = jnp.bitwise_and(cb, PACK - 1)
                for k in range(L):
                    e = gbase + k
                    row = jnp.where(e < cnt, d16[k] - lo, NR)
                    er = q * L + k
                    for j in range(NCHUNK // 2):
                        sl = pl.ds(j * L, L)
                        w32 = rows_v[er, sl]
                        wlo = jax.lax.bitcast_convert_type(
                            jax.lax.shift_left(w32, 16), jnp.float32)
                        whi = jax.lax.bitcast_convert_type(
                            jnp.bitwise_and(w32, -65536), jnp.float32)
                        slh = pl.ds((NCHUNK // 2 + j) * L, L)
                        acc_v[row, sl] = acc_v[row, sl] + wlo
                        acc_v[row, slh] = acc_v[row, slh] + whi
                    dega_sm[row] = dega_sm[row] + 1.0
                return 0

            lax.fori_loop(0, GB // L, grp, 0)
            return 0

        lax.fori_loop(0, nb, gblk, 0)
        return 0

    lax.fori_loop(0, NSB, sblk, 0)

    def fin(i, _):
        degv[pl.ds(i, L)] = one * dega_sm[i]
        return 0

    lax.fori_loop(0, NR, fin, 0)
    pltpu.sync_copy(acc_v.at[pl.ds(0, NR)], agg_hbm.at[pl.ds(w * NR, NR)])
    pltpu.sync_copy(degv.at[pl.ds(0, NR)], deg_hbm.at[pl.ds(w * NR, NR)])


def _emb_body(x_ref, agg_ref, deg_ref, ws_ref, wn_ref, h_ref):
    deg = deg_ref[...]
    scale = 1.0 / jnp.maximum(deg, 1.0)
    a = agg_ref[...] * scale
    h = jnp.dot(x_ref[...], ws_ref[...], preferred_element_type=jnp.float32)
    h = h + jnp.dot(a, wn_ref[...], preferred_element_type=jnp.float32)
    h_ref[...] = jnp.maximum(h, 0.0).astype(jnp.bfloat16)


_ROWS_BLK = 1000


def _emb(x, agg, degw, W_self, W_neigh):
    return pl.pallas_call(
        _emb_body,
        grid=(N // _ROWS_BLK,),
        in_specs=[
            pl.BlockSpec((_ROWS_BLK, D), lambda i: (i, 0)),
            pl.BlockSpec((_ROWS_BLK, D), lambda i: (i, 0)),
            pl.BlockSpec((_ROWS_BLK, 1), lambda i: (i, 0)),
            pl.BlockSpec((D, D), lambda i: (0, 0)),
            pl.BlockSpec((D, D), lambda i: (0, 0)),
        ],
        out_specs=pl.BlockSpec((_ROWS_BLK, D), lambda i: (i, 0)),
        out_shape=jax.ShapeDtypeStruct((N, D), jnp.bfloat16),
    )(x, agg, degw, W_self, W_neigh)


DH = D // 2


@functools.partial(
    pl.kernel,
    out_type=[
        jax.ShapeDtypeStruct((E,), jnp.float32),
        jax.ShapeDtypeStruct((E,), jnp.float32),
    ],
    mesh=_MESH,
    scratch_types=[
        pltpu.VMEM((EB, DH), jnp.int32),
        pltpu.VMEM((EB, DH), jnp.int32),
        pltpu.VMEM((EB, DH), jnp.int32),
        pltpu.VMEM((EB, DH), jnp.int32),
        pltpu.VMEM((EB,), jnp.int32),
        pltpu.VMEM((EB,), jnp.int32),
        pltpu.VMEM((EB,), jnp.int32),
        pltpu.VMEM((EB,), jnp.int32),
        pltpu.VMEM((DH,), jnp.float32),
        pltpu.VMEM((DH,), jnp.float32),
        pltpu.VMEM((EPT,), jnp.float32),
        pltpu.SemaphoreType.DMA,
        pltpu.SemaphoreType.DMA,
        pltpu.SemaphoreType.DMA,
        pltpu.SemaphoreType.DMA,
    ],
)
def _score(h_hbm, pos_hbm, neg_hbm, re_hbm, ro_hbm, pos_out, neg_out,
           ua, va, ub, vb, uia, via, uib, vib, re_v, ro_v, sbuf,
           sua, sva, sub_, svb):
    c = lax.axis_index("c")
    s = lax.axis_index("s")

    pltpu.sync_copy(re_hbm, re_v)
    pltpu.sync_copy(ro_hbm, ro_v)
    re_regs = [re_v[pl.ds(j * L, L)] for j in range(DH // L)]
    ro_regs = [ro_v[pl.ds(j * L, L)] for j in range(DH // L)]
    lane = lax.iota(jnp.int32, L)
    onehots = [jnp.where(lane == k, 1.0, 0.0) for k in range(L)]

    def do_set(ei_hbm, out_hbm):
        def fire(b, ui, vi, ur, vr, su, sv):
            base = s * EPT + b * EB
            pltpu.sync_copy(ei_hbm.at[pl.ds(base, EB)], ui)
            pltpu.sync_copy(ei_hbm.at[pl.ds(E + base, EB)], vi)
            cu = pltpu.async_copy(h_hbm.at[ui], ur, su)
            cv = pltpu.async_copy(h_hbm.at[vi], vr, sv)
            return cu, cv

        def compute(b, ur, vr):
            def grp(q, _):
                vec = jnp.zeros((L,), jnp.float32)
                for k in range(L):
                    e = q * L + k
                    acc = None
                    for j in range(DH // L):
                        sl = pl.ds(j * L, L)
                        uw = ur[e, sl]
                        vw = vr[e, sl]
                        ulo = jax.lax.bitcast_convert_type(
                            jax.lax.shift_left(uw, 16), jnp.float32)
                        uhi = jax.lax.bitcast_convert_type(uw, jnp.float32)
                        vlo = jax.lax.bitcast_convert_type(
                            jax.lax.shift_left(vw, 16), jnp.float32)
                        vhi = jax.lax.bitcast_convert_type(vw, jnp.float32)
                        t = ulo * vlo * re_regs[j] + uhi * vhi * ro_regs[j]
                        acc = t if acc is None else acc + t
                    lanes = [acc[i] for i in range(L)]
                    while len(lanes) > 1:
                        lanes = [lanes[i] + lanes[i + 1]
                                 for i in range(0, len(lanes), 2)]
                    vec = vec + lanes[0] * onehots[k]
                sbuf[pl.ds(b * EB + q * L, L)] = vec
                return 0

            lax.fori_loop(0, EB // L, grp, 0)

        ca = fire(0, uia, via, ua, va, sua, sva)

        def wait(ur, su):
            pltpu.make_async_copy(h_hbm.at[uia], ur, su).wait()

        def pair(i, _):
            b = 2 * i
            wait(ua, sua)
            wait(va, sva)
            fire(b + 1, uib, vib, ub, vb, sub_, svb)
            compute(b, ua, va)
            wait(ub, sub_)
            wait(vb, svb)
            fire(b + 2, uia, via, ua, va, sua, sva)
            compute(b + 1, ub, vb)
            return 0

        lax.fori_loop(0, (NBLK - 1) // 2, pair, 0)
        wait(ua, sua)
        wait(va, sva)
        compute(NBLK - 1, ua, va)
        pltpu.sync_copy(sbuf, out_hbm.at[pl.ds(s * EPT, EPT)])

    @pl.when(c == 0)
    def _():
        do_set(pos_hbm, pos_out)

    @pl.when(c == 1)
    def _():
        do_set(neg_hbm, neg_out)


def kernel(x, block_edge_index, pos_edge_index, neg_edge_index, W_self, W_neigh, r):
    x32 = jax.lax.bitcast_convert_type(
        x.astype(jnp.bfloat16).reshape(N, D // 2, 2), jnp.int32)
    agg, degw = _seg_sum(x32, block_edge_index.reshape(-1))
    Wn_perm = jnp.concatenate([W_neigh[0::2], W_neigh[1::2]], axis=0)
    h = _emb(x, agg[:N], degw[:N].reshape(N, 1), W_self, Wn_perm)
    h32 = jax.lax.bitcast_convert_type(h.reshape(N, DH, 2), jnp.int32)
    pos_score, neg_score = _score(h32, pos_edge_index.reshape(-1),
                                  neg_edge_index.reshape(-1),
                                  r[0::2], r[1::2])
    return (pos_score, neg_score)


# speedup vs baseline: 1.1621x; 1.0887x over previous
import functools

import jax
import jax.numpy as jnp
from jax import lax
from jax.experimental import pallas as pl
from jax.experimental.pallas import tpu as pltpu
from jax.experimental.pallas import tpu_sc as plsc

N = 10000
E = 160000
D = 256

NC = 2
NS = 16
L = 16
NW = NC * NS

NCHUNK = D // L
NR = 320
NPAD = NW * NR
ACC_R = NR + 1
SCB = 8000
NSB = E // SCB
CAP = SCB + L
GB = 80
PACK = 16384

EB = 80
EPT = E // NS
NBLK = EPT // EB

_MESH = plsc.VectorSubcoreMesh(core_axis_name="c", subcore_axis_name="s")


@functools.partial(
    pl.kernel,
    out_type=[
        jax.ShapeDtypeStruct((NPAD, D), jnp.float32),
        jax.ShapeDtypeStruct((NPAD,), jnp.float32),
    ],
    mesh=_MESH,
    scratch_types=[
        pltpu.VMEM((GB, D // 2), jnp.int32),
        pltpu.VMEM((ACC_R, D), jnp.float32),
        pltpu.VMEM((NR + L,), jnp.float32),
        pltpu.SMEM((ACC_R,), jnp.float32),
        pltpu.VMEM((CAP,), jnp.int32),
        pltpu.VMEM((GB,), jnp.int32),
        pltpu.VMEM((SCB,), jnp.int32),
        pltpu.VMEM((SCB,), jnp.int32),
        pltpu.SemaphoreType.DMA,
    ],
)
def _seg_sum(x_hbm, bei_hbm, agg_hbm, deg_hbm,
             rows_v, acc_v, degv, dega_sm, idxc, sg_v, src_v, dst_v, sem):
    c = lax.axis_index("c")
    s = lax.axis_index("s")
    w = c * NS + s
    lo = w * NR

    zero = jnp.zeros((L,), jnp.float32)
    zero_i = jnp.zeros((L,), jnp.int32)
    one = jnp.ones((L,), jnp.float32)
    ones_i = jnp.ones((L,), jnp.int32)

    def z_acc(i, _):
        for j in range(NCHUNK):
            acc_v[i, pl.ds(j * L, L)] = zero
        dega_sm[i] = 0.0
        return 0

    lax.fori_loop(0, ACC_R, z_acc, 0)

    def z_deg(i, _):
        degv[pl.ds(i * L, L)] = zero
        return 0

    lax.fori_loop(0, (NR + L) // L, z_deg, 0)

    def z_idx(i, _):
        idxc[pl.ds(i * L, L)] = zero_i
        return 0

    lax.fori_loop(0, CAP // L, z_idx, 0)
    for k2 in range(GB // L):
        sg_v[pl.ds(k2 * L, L)] = zero_i

    def sblk(b, _):
        ebase = b * SCB
        pltpu.sync_copy(bei_hbm.at[pl.ds(ebase, SCB)], src_v)
        pltpu.sync_copy(bei_hbm.at[pl.ds(E + ebase, SCB)], dst_v)

        def chunk(t, cnt):
            s16 = src_v[pl.ds(t * L, L)]
            d16 = dst_v[pl.ds(t * L, L)]
            comb = s16 * PACK + d16
            okv = (d16 >= lo) & (d16 < lo + NR)
            oki = jnp.where(okv, 1, 0)
            for k in range(L):
                idxc[pl.ds(cnt, L)] = ones_i * comb[k]
                cnt = cnt + oki[k]
            return cnt

        cnt = lax.fori_loop(0, SCB // L, chunk, jnp.int32(0))

        nb = (cnt + (GB - 1)) // GB

        def gblk(bb, _):
            for k2 in range(GB // L):
                cb0 = idxc[pl.ds(bb * GB + k2 * L, L)]
                sg_v[pl.ds(k2 * L, L)] = jnp.right_shift(cb0, 14)
            pltpu.async_copy(x_hbm.at[sg_v], rows_v, sem).wait()

            def grp(q, _):
                gbase = bb * GB + q * L
                cb = idxc[pl.ds(gbase, L)]
                d16 = jnp.bitwise_and(cb, PACK - 1)
                for k in range(L):
                    e = gbase + k
                    row = jnp.where(e < cnt, d16[k] - lo, NR)
                    er = q * L + k
                    for j in range(NCHUNK // 2):
                        sl = pl.ds(j * L, L)
                        w32 = rows_v[er, sl]
                        wlo = jax.lax.bitcast_convert_type(
                            jax.lax.shift_left(w32, 16), jnp.float32)
                        whi = jax.lax.bitcast_convert_type(
                            jnp.bitwise_and(w32, -65536), jnp.float32)
                        slh = pl.ds((NCHUNK // 2 + j) * L, L)
                        acc_v[row, sl] = acc_v[row, sl] + wlo
                        acc_v[row, slh] = acc_v[row, slh] + whi
                    dega_sm[row] = dega_sm[row] + 1.0
                return 0

            lax.fori_loop(0, GB // L, grp, 0)
            return 0

        lax.fori_loop(0, nb, gblk, 0)
        return 0

    lax.fori_loop(0, NSB, sblk, 0)

    def fin(i, _):
        degv[pl.ds(i, L)] = one * dega_sm[i]
        return 0

    lax.fori_loop(0, NR, fin, 0)
    pltpu.sync_copy(acc_v.at[pl.ds(0, NR)], agg_hbm.at[pl.ds(w * NR, NR)])
    pltpu.sync_copy(degv.at[pl.ds(0, NR)], deg_hbm.at[pl.ds(w * NR, NR)])


def _emb_body(x_ref, agg_ref, deg_ref, ws_ref, wn_ref, h_ref):
    deg = deg_ref[...]
    scale = 1.0 / jnp.maximum(deg, 1.0)
    a = agg_ref[...] * scale
    h = jnp.dot(x_ref[...], ws_ref[...], preferred_element_type=jnp.float32)
    h = h + jnp.dot(a, wn_ref[...], preferred_element_type=jnp.float32)
    h_ref[...] = jnp.maximum(h, 0.0).astype(jnp.bfloat16)


_ROWS_BLK = 1000


def _emb(x, agg, degw, W_self, W_neigh):
    return pl.pallas_call(
        _emb_body,
        grid=(N // _ROWS_BLK,),
        in_specs=[
            pl.BlockSpec((_ROWS_BLK, D), lambda i: (i, 0)),
            pl.BlockSpec((_ROWS_BLK, D), lambda i: (i, 0)),
            pl.BlockSpec((_ROWS_BLK, 1), lambda i: (i, 0)),
            pl.BlockSpec((D, D), lambda i: (0, 0)),
            pl.BlockSpec((D, D), lambda i: (0, 0)),
        ],
        out_specs=pl.BlockSpec((_ROWS_BLK, D), lambda i: (i, 0)),
        out_shape=jax.ShapeDtypeStruct((N, D), jnp.bfloat16),
    )(x, agg, degw, W_self, W_neigh)


DH = D // 2


@functools.partial(
    pl.kernel,
    out_type=[
        jax.ShapeDtypeStruct((E,), jnp.float32),
        jax.ShapeDtypeStruct((E,), jnp.float32),
    ],
    mesh=_MESH,
    scratch_types=[
        pltpu.VMEM((EB, DH), jnp.int32),
        pltpu.VMEM((EB, DH), jnp.int32),
        pltpu.VMEM((EB, DH), jnp.int32),
        pltpu.VMEM((EB, DH), jnp.int32),
        pltpu.VMEM((EPT,), jnp.int32),
        pltpu.VMEM((EPT,), jnp.int32),
        pltpu.VMEM((DH,), jnp.float32),
        pltpu.VMEM((DH,), jnp.float32),
        pltpu.VMEM((EPT,), jnp.float32),
        pltpu.SemaphoreType.DMA,
        pltpu.SemaphoreType.DMA,
        pltpu.SemaphoreType.DMA,
        pltpu.SemaphoreType.DMA,
    ],
)
def _score(h_hbm, pos_hbm, neg_hbm, re_hbm, ro_hbm, pos_out, neg_out,
           ua, va, ub, vb, uall, vall, re_v, ro_v, sbuf,
           sua, sva, sub_, svb):
    c = lax.axis_index("c")
    s = lax.axis_index("s")

    pltpu.sync_copy(re_hbm, re_v)
    pltpu.sync_copy(ro_hbm, ro_v)
    re_regs = [re_v[pl.ds(j * L, L)] for j in range(DH // L)]
    ro_regs = [ro_v[pl.ds(j * L, L)] for j in range(DH // L)]
    lane = lax.iota(jnp.int32, L)
    onehots = [jnp.where(lane == k, 1.0, 0.0) for k in range(L)]

    def do_set(ei_hbm, out_hbm):
        pltpu.sync_copy(ei_hbm.at[pl.ds(s * EPT, EPT)], uall)
        pltpu.sync_copy(ei_hbm.at[pl.ds(E + s * EPT, EPT)], vall)

        def fire(b, ur, vr, su, sv):
            cu = pltpu.async_copy(h_hbm.at[uall.at[pl.ds(b * EB, EB)]], ur, su)
            cv = pltpu.async_copy(h_hbm.at[vall.at[pl.ds(b * EB, EB)]], vr, sv)
            return cu, cv

        def compute(b, ur, vr):
            def grp(q, _):
                vec = jnp.zeros((L,), jnp.float32)
                for k in range(L):
                    e = q * L + k
                    acc = None
                    for j in range(DH // L):
                        sl = pl.ds(j * L, L)
                        uw = ur[e, sl]
                        vw = vr[e, sl]
                        ulo = jax.lax.bitcast_convert_type(
                            jax.lax.shift_left(uw, 16), jnp.float32)
                        uhi = jax.lax.bitcast_convert_type(uw, jnp.float32)
                        vlo = jax.lax.bitcast_convert_type(
                            jax.lax.shift_left(vw, 16), jnp.float32)
                        vhi = jax.lax.bitcast_convert_type(vw, jnp.float32)
                        t = ulo * vlo * re_regs[j] + uhi * vhi * ro_regs[j]
                        acc = t if acc is None else acc + t
                    lanes = [acc[i] for i in range(L)]
                    while len(lanes) > 1:
                        lanes = [lanes[i] + lanes[i + 1]
                                 for i in range(0, len(lanes), 2)]
                    vec = vec + lanes[0] * onehots[k]
                sbuf[pl.ds(b * EB + q * L, L)] = vec
                return 0

            lax.fori_loop(0, EB // L, grp, 0)

        ca = fire(0, ua, va, sua, sva)

        def wait(ur, su):
            pltpu.make_async_copy(h_hbm.at[uall.at[pl.ds(0, EB)]], ur, su).wait()

        def pair(i, _):
            b = 2 * i
            wait(ua, sua)
            wait(va, sva)
            fire(b + 1, ub, vb, sub_, svb)
            compute(b, ua, va)
            wait(ub, sub_)
            wait(vb, svb)
            fire(b + 2, ua, va, sua, sva)
            compute(b + 1, ub, vb)
            return 0

        lax.fori_loop(0, (NBLK - 1) // 2, pair, 0)
        wait(ua, sua)
        wait(va, sva)
        compute(NBLK - 1, ua, va)
        pltpu.sync_copy(sbuf, out_hbm.at[pl.ds(s * EPT, EPT)])

    @pl.when(c == 0)
    def _():
        do_set(pos_hbm, pos_out)

    @pl.when(c == 1)
    def _():
        do_set(neg_hbm, neg_out)


def kernel(x, block_edge_index, pos_edge_index, neg_edge_index, W_self, W_neigh, r):
    x32 = jax.lax.bitcast_convert_type(
        x.astype(jnp.bfloat16).reshape(N, D // 2, 2), jnp.int32)
    agg, degw = _seg_sum(x32, block_edge_index.reshape(-1))
    Wn_perm = jnp.concatenate([W_neigh[0::2], W_neigh[1::2]], axis=0)
    h = _emb(x, agg[:N], degw[:N].reshape(N, 1), W_self, Wn_perm)
    h32 = jax.lax.bitcast_convert_type(h.reshape(N, DH, 2), jnp.int32)
    pos_score, neg_score = _score(h32, pos_edge_index.reshape(-1),
                                  neg_edge_index.reshape(-1),
                                  r[0::2], r[1::2])
    return (pos_score, neg_score)
